# baseline (device time: 13322 ns/iter reference)
import jax
import jax.numpy as jnp
from jax import lax
from jax.experimental import pallas as pl
from jax.experimental.pallas import tpu as pltpu

Y = 4
T = 256
V_SH = 4096


def kernel(x, W, labels):
    t, d = x.shape
    v_sh = W.shape[1]

    def body(x_ref, w_ref, labels_ref, out_ref,
             local_ref, recv_ref, send_sems, recv_sems):
        my_x = lax.axis_index("x")
        my_y = lax.axis_index("y")
        my_z = lax.axis_index("z")

        barrier_sem = pltpu.get_barrier_semaphore()
        for j in range(1, Y):
            pl.semaphore_signal(
                barrier_sem, inc=1,
                device_id=(my_x, (my_y + j) % Y, my_z),
                device_id_type=pl.DeviceIdType.MESH,
            )
        pl.semaphore_wait(barrier_sem, Y - 1)

        logits = jnp.dot(x_ref[...].astype(jnp.bfloat16),
                         w_ref[...].astype(jnp.bfloat16),
                         preferred_element_type=jnp.float32)
        m = jnp.max(logits, axis=1)
        s = jnp.sum(jnp.exp(logits - m[:, None]), axis=1)
        cols = lax.broadcasted_iota(jnp.int32, (t, v_sh), 1) + my_y * v_sh
        hit = cols == labels_ref[...][:, None]
        tl = jnp.sum(jnp.where(hit, logits, 0.0), axis=1)

        local_ref[0, :] = m
        local_ref[1, :] = s
        local_ref[2, :] = tl

        COMM = False
        rdmas = []
        for j in range(1, Y) if COMM else []:
            rdma = pltpu.make_async_remote_copy(
                src_ref=local_ref,
                dst_ref=recv_ref.at[Y - j - 1],
                send_sem=send_sems.at[j - 1],
                recv_sem=recv_sems.at[Y - j - 1],
                device_id=(my_x, (my_y + j) % Y, my_z),
                device_id_type=pl.DeviceIdType.MESH,
            )
            rdma.start()
            rdmas.append(rdma)
        for rdma in rdmas:
            rdma.wait_send()
            rdma.wait_recv()

        big_m = m
        for k in range(Y - 1):
            big_m = jnp.maximum(big_m, recv_ref[k, 0, :])
        acc_s = s * jnp.exp(m - big_m)
        acc_tl = tl
        for k in range(Y - 1):
            acc_s = acc_s + recv_ref[k, 1, :] * jnp.exp(recv_ref[k, 0, :] - big_m)
            acc_tl = acc_tl + recv_ref[k, 2, :]

        out_ref[...] = big_m + jnp.log(acc_s) - acc_tl

    return pl.pallas_call(
        body,
        out_shape=jax.ShapeDtypeStruct((t,), jnp.float32),
        in_specs=[
            pl.BlockSpec(memory_space=pltpu.VMEM),
            pl.BlockSpec(memory_space=pltpu.VMEM),
            pl.BlockSpec(memory_space=pltpu.VMEM),
        ],
        out_specs=pl.BlockSpec(memory_space=pltpu.VMEM),
        scratch_shapes=[
            pltpu.VMEM((3, t), jnp.float32),
            pltpu.VMEM((Y - 1, 3, t), jnp.float32),
            pltpu.SemaphoreType.DMA((Y - 1,)),
            pltpu.SemaphoreType.DMA((Y - 1,)),
        ],
        compiler_params=pltpu.CompilerParams(collective_id=0),
    )(x, W, labels)


# device time: 11646 ns/iter; 1.1439x vs baseline; 1.1439x over previous
import jax
import jax.numpy as jnp
from jax import lax
from jax.experimental import pallas as pl
from jax.experimental.pallas import tpu as pltpu

Y = 4
SPLIT_X = 2
NGROUP = Y * SPLIT_X

OFFSETS = [(dx, dy) for dx in range(SPLIT_X) for dy in range(Y)]


def _neg(off):
    dx, dy = off
    return ((-dx) % 2, (-dy) % Y)


def kernel(x, W, labels):
    t, d = x.shape
    v_sh = W.shape[1]
    chunk = v_sh // SPLIT_X

    def body(x_hbm, w_hbm, labels_hbm, out_hbm,
             x_ref, w_ref, labels_ref, out_vmem, recv_ref,
             send_sems, recv_sems, cp_sems):
        my_x = lax.axis_index("x")
        my_y = lax.axis_index("y")
        my_z = lax.axis_index("z")
        gco = my_y * v_sh + my_x * chunk

        cp_x = pltpu.make_async_copy(x_hbm, x_ref, cp_sems.at[0])
        cp_x.start()
        cp_l = pltpu.make_async_copy(labels_hbm, labels_ref, cp_sems.at[1])
        cp_l.start()
        cp_w = pltpu.make_async_copy(
            w_hbm.at[:, pl.ds(my_x * chunk, chunk)], w_ref, cp_sems.at[2])
        cp_w.start()

        barrier_sem = pltpu.get_barrier_semaphore()
        for dx, dy in OFFSETS[1:]:
            pl.semaphore_signal(
                barrier_sem, inc=1,
                device_id=((my_x + dx) % 2, (my_y + dy) % Y, my_z),
                device_id_type=pl.DeviceIdType.MESH,
            )

        cp_x.wait()
        cp_l.wait()
        cp_w.wait()

        logits = jnp.dot(x_ref[...], w_ref[...],
                         preferred_element_type=jnp.float32)
        m = jnp.max(logits, axis=1)
        s = jnp.sum(jnp.exp(logits - m[:, None]), axis=1)
        cols = lax.broadcasted_iota(jnp.int32, (t, chunk), 1) + gco
        hit = cols == labels_ref[...][:, None]
        tl = jnp.sum(jnp.where(hit, logits, 0.0), axis=1)

        recv_ref[0, 0, :] = m
        recv_ref[0, 1, :] = s
        recv_ref[0, 2, :] = tl

        pl.semaphore_wait(barrier_sem, NGROUP - 1)

        rdmas = []
        for k, off in enumerate(OFFSETS[1:], start=1):
            dx, dy = off
            kslot = OFFSETS.index(_neg(off))
            rdma = pltpu.make_async_remote_copy(
                src_ref=recv_ref.at[0],
                dst_ref=recv_ref.at[kslot],
                send_sem=send_sems.at[k - 1],
                recv_sem=recv_sems.at[kslot],
                device_id=((my_x + dx) % 2, (my_y + dy) % Y, my_z),
                device_id_type=pl.DeviceIdType.MESH,
            )
            rdma.start()
            rdmas.append(rdma)
        for rdma in rdmas:
            rdma.wait_send()
        for rdma in rdmas:
            rdma.wait_recv()

        all_m = recv_ref[:, 0, :]
        big_m = jnp.max(all_m, axis=0)
        acc_s = jnp.sum(recv_ref[:, 1, :] * jnp.exp(all_m - big_m[None, :]),
                        axis=0)
        acc_tl = jnp.sum(recv_ref[:, 2, :], axis=0)
        out_vmem[...] = big_m + jnp.log(acc_s) - acc_tl

        cp_o = pltpu.make_async_copy(out_vmem, out_hbm, cp_sems.at[3])
        cp_o.start()
        cp_o.wait()

    return pl.pallas_call(
        body,
        out_shape=jax.ShapeDtypeStruct((t,), jnp.float32),
        in_specs=[pl.BlockSpec(memory_space=pl.ANY)] * 3,
        out_specs=pl.BlockSpec(memory_space=pl.ANY),
        scratch_shapes=[
            pltpu.VMEM((t, d), jnp.float32),
            pltpu.VMEM((d, chunk), jnp.float32),
            pltpu.VMEM((t,), jnp.int32),
            pltpu.VMEM((t,), jnp.float32),
            pltpu.VMEM((NGROUP, 3, t), jnp.float32),
            pltpu.SemaphoreType.DMA((NGROUP - 1,)),
            pltpu.SemaphoreType.DMA((NGROUP,)),
            pltpu.SemaphoreType.DMA((4,)),
        ],
        compiler_params=pltpu.CompilerParams(collective_id=0),
    )(pltpu.with_memory_space_constraint(x, pltpu.MemorySpace.HBM),
      pltpu.with_memory_space_constraint(W, pltpu.MemorySpace.HBM),
      pltpu.with_memory_space_constraint(labels, pltpu.MemorySpace.HBM))
